# baseline (speedup 1.0000x reference)
import functools

import jax
import jax.numpy as jnp
from jax import lax
from jax.experimental import pallas as pl
from jax.experimental.pallas import tpu as pltpu
from jax.experimental.pallas import tpu_sc as plsc

N = 50000
NB = 50000
E = 800000
R = 1000
D_FEAT = 100
D_REL = 300
D_HID = 200

C = 32
K = 8
KH = 4
DP = K * C
DEF = 128
NTAB = 50176
NBQ = 57344
EP = 811008
DUMMY = 50000
B = 128
SUP = 2

STRIPE = NTAB // 16
RPT32 = NBQ // 32
EBT = EP // (16 * B)
NBT = NBQ // (16 * B)

_MESH = plsc.VectorSubcoreMesh(core_axis_name="c", subcore_axis_name="s")
_SC_PARAMS = pltpu.CompilerParams(use_tc_tiling_on_sc=False)
_RB = 512
_GRID = NTAB // _RB


def _t0_body(rel_ref, wr_ref, br_ref, w1b_ref, outa_ref, outb_ref):
    rc = jnp.dot(rel_ref[...], wr_ref[...], preferred_element_type=jnp.float32)
    rc = jnp.maximum(rc + br_ref[...], 0.0)
    s = jnp.dot(rc, w1b_ref[...], preferred_element_type=jnp.float32)
    outa_ref[...] = s[:, :128]
    outb_ref[...] = s[:, 128:]


def _t0(rel, wr, br, w1b):
    return pl.pallas_call(
        _t0_body,
        out_shape=(jax.ShapeDtypeStruct((R, 128), jnp.float32),
                   jax.ShapeDtypeStruct((R, 128), jnp.float32)),
    )(rel, wr, br, w1b)


def _t2_body(ef_ref, s1a_ref, s1b_ref, w_ref, outa_ref, outb_ref):
    s = jnp.dot(ef_ref[...], w_ref[...], preferred_element_type=jnp.float32)
    outa_ref[...] = s[:, :128] + s1a_ref[...]
    outb_ref[...] = s[:, 128:] + s1b_ref[...]


def _t2(ef, s1a, s1b, w):
    return pl.pallas_call(
        _t2_body,
        grid=(_GRID,),
        in_specs=[
            pl.BlockSpec((_RB, DEF), lambda i: (i, 0)),
            pl.BlockSpec((_RB, 128), lambda i: (i, 0)),
            pl.BlockSpec((_RB, 128), lambda i: (i, 0)),
            pl.BlockSpec((DEF, DP), lambda i: (0, 0)),
        ],
        out_specs=(pl.BlockSpec((_RB, 128), lambda i: (i, 0)),
                   pl.BlockSpec((_RB, 128), lambda i: (i, 0))),
        out_shape=(jax.ShapeDtypeStruct((NTAB, 128), jnp.float32),
                   jax.ShapeDtypeStruct((NTAB, 128), jnp.float32)),
    )(ef, s1a, s1b, w)


def _t3_body(deg_ref, cnt_ref, invd_ref, invc_ref):
    def unpack(x):
        parts = [jnp.broadcast_to(x[:, 32 * a:32 * a + 1], (_RB // 4, 128))
                 for a in range(4)]
        return jnp.stack(parts, axis=1).reshape(_RB, 128)

    invd_ref[...] = 1.0 / (unpack(deg_ref[...]) + 1.0)
    invc_ref[...] = 1.0 / jnp.maximum(unpack(cnt_ref[...]), 1.0)


def _t3(deg4, cnt4):
    return pl.pallas_call(
        _t3_body,
        grid=(_GRID,),
        in_specs=[
            pl.BlockSpec((_RB // 4, 128), lambda i: (i, 0)),
            pl.BlockSpec((_RB // 4, 128), lambda i: (i, 0)),
        ],
        out_specs=(pl.BlockSpec((_RB, 128), lambda i: (i, 0)),
                   pl.BlockSpec((_RB, 128), lambda i: (i, 0))),
        out_shape=(jax.ShapeDtypeStruct((NTAB, 128), jnp.float32),
                   jax.ShapeDtypeStruct((NTAB, 128), jnp.float32)),
    )(deg4, cnt4)


def _t4_body(agga_ref, aggb_ref, supa_ref, supb_ref, inv_ref, w_ref,
             outa_ref, outb_ref):
    inv = inv_ref[...]
    ha = jnp.maximum((agga_ref[...] + supa_ref[...]) * inv, 0.0)
    hb = jnp.maximum((aggb_ref[...] + supb_ref[...]) * inv, 0.0)
    h = jnp.concatenate([ha, hb], axis=1)
    s2 = jnp.dot(h, w_ref[...], preferred_element_type=jnp.float32)
    outa_ref[...] = s2[:, :128]
    outb_ref[...] = s2[:, 128:]


def _t4(agga, aggb, supa, supb, inv, w):
    bs = pl.BlockSpec((_RB, 128), lambda i: (i, 0))
    return pl.pallas_call(
        _t4_body,
        grid=(_GRID,),
        in_specs=[bs, bs, bs, bs, bs, pl.BlockSpec((DP, DP), lambda i: (0, 0))],
        out_specs=(bs, bs),
        out_shape=(jax.ShapeDtypeStruct((NTAB, 128), jnp.float32),
                   jax.ShapeDtypeStruct((NTAB, 128), jnp.float32)),
    )(agga, aggb, supa, supb, inv, w)


def _t6_body(agga_ref, aggb_ref, supa_ref, supb_ref, inv_ref,
             outa_ref, outb_ref):
    inv = inv_ref[...]
    outa_ref[...] = (agga_ref[...] + supa_ref[...]) * inv
    outb_ref[...] = (aggb_ref[...] + supb_ref[...]) * inv


def _t6(agga, aggb, supa, supb, inv):
    bs = pl.BlockSpec((_RB, 128), lambda i: (i, 0))
    return pl.pallas_call(
        _t6_body,
        grid=(_GRID,),
        in_specs=[bs, bs, bs, bs, bs],
        out_specs=(bs, bs),
        out_shape=(jax.ShapeDtypeStruct((NBQ, 128), jnp.float32),
                   jax.ShapeDtypeStruct((NBQ, 128), jnp.float32)),
    )(agga, aggb, supa, supb, inv)


def _t8_body(sumsa_ref, sumsb_ref, inv_ref, outa_ref, outb_ref):
    inv = inv_ref[...]
    outa_ref[...] = sumsa_ref[...] * inv
    outb_ref[...] = sumsb_ref[...] * inv


def _t8(sumsa, sumsb, inv):
    bs = pl.BlockSpec((_RB, 128), lambda i: (i, 0))
    return pl.pallas_call(
        _t8_body,
        grid=(_GRID,),
        in_specs=[bs, bs, bs],
        out_specs=(bs, bs),
        out_shape=(jax.ShapeDtypeStruct((NTAB, 128), jnp.float32),
                   jax.ShapeDtypeStruct((NTAB, 128), jnp.float32)),
    )(sumsa, sumsb, inv)


@functools.partial(
    pl.kernel,
    compiler_params=_SC_PARAMS,
    out_type=(jax.ShapeDtypeStruct((NBQ, DEF), jnp.float32),
              jax.ShapeDtypeStruct((NBQ, 128), jnp.float32),
              jax.ShapeDtypeStruct((NBQ, 128), jnp.float32)),
    mesh=_MESH,
    scratch_types=[
        pltpu.VMEM((B,), jnp.int32),
        pltpu.VMEM((B,), jnp.int32),
        pltpu.VMEM((B,), jnp.int32),
        pltpu.VMEM((B,), jnp.int32),
        pltpu.VMEM((B, DEF), jnp.float32),
        pltpu.VMEM((B, DEF), jnp.float32),
        pltpu.VMEM((B, 128), jnp.float32),
        pltpu.VMEM((B, 128), jnp.float32),
        pltpu.VMEM((B, 128), jnp.float32),
        pltpu.VMEM((B, 128), jnp.float32),
        pltpu.SemaphoreType.DMA,
        pltpu.SemaphoreType.DMA,
    ],
)
def _s1(bx_hbm, bgi_hbm, ef_hbm, rcwa_hbm, rcwb_hbm, ef_out, s1a_out, s1b_out,
        ix0, ix1, ig0, ig1, re0, re1, ra0, ra1, rb0, rb1, sem0, sem1):
    wid = lax.axis_index("s") * 2 + lax.axis_index("c")
    base0 = wid * RPT32
    nb = RPT32 // B
    ix = (ix0, ix1)
    ig = (ig0, ig1)
    re = (re0, re1)
    ra = (ra0, ra1)
    rb = (rb0, rb1)
    sem = (sem0, sem1)

    def fire(b, p):
        blk = pl.ds(base0 + b * B, B)
        pltpu.sync_copy(bx_hbm.at[blk], ix[p])
        pltpu.sync_copy(bgi_hbm.at[blk], ig[p])
        pltpu.async_copy(ef_hbm.at[ix[p]], re[p], sem[p])
        pltpu.async_copy(rcwa_hbm.at[ig[p]], ra[p], sem[p])
        pltpu.async_copy(rcwb_hbm.at[ig[p]], rb[p], sem[p])

    def drain(b, p):
        pltpu.make_async_copy(ef_hbm.at[ix[p]], re[p], sem[p]).wait()
        pltpu.make_async_copy(rcwa_hbm.at[ig[p]], ra[p], sem[p]).wait()
        pltpu.make_async_copy(rcwb_hbm.at[ig[p]], rb[p], sem[p]).wait()
        blk = pl.ds(base0 + b * B, B)
        pltpu.sync_copy(re[p], ef_out.at[blk])
        pltpu.sync_copy(ra[p], s1a_out.at[blk])
        pltpu.sync_copy(rb[p], s1b_out.at[blk])

    fire(0, 0)

    @pl.loop(0, nb, step=2)
    def _(b):
        i_last = (b + 2) >= nb
        fire(b + 1, 1)
        drain(b, 0)

        @pl.when(jnp.logical_not(i_last))
        def _():
            fire(b + 2, 0)

        drain(b + 1, 1)


@functools.partial(
    pl.kernel,
    compiler_params=_SC_PARAMS,
    out_type=(jax.ShapeDtypeStruct((NTAB, C), jnp.float32),
              jax.ShapeDtypeStruct((NTAB, C), jnp.float32)),
    mesh=_MESH,
    scratch_types=[
        pltpu.VMEM((B,), jnp.int32),
        pltpu.VMEM((B, C), jnp.float32),
        pltpu.VMEM_SHARED((NTAB, C), jnp.float32),
    ],
)
def _s3a(dst_hbm, bx_hbm, ones_hbm, zeros_hbm, deg_out, cnt_out,
         didx, ones_v, acc):
    cid = lax.axis_index("c")
    sid = lax.axis_index("s")
    stripe_slc = pl.ds(sid * STRIPE, STRIPE)
    pltpu.sync_copy(ones_hbm, ones_v)
    pltpu.sync_copy(zeros_hbm, acc.at[stripe_slc])
    plsc.subcore_barrier()

    @pl.when(cid == 0)
    def _():
        @pl.loop(0, EBT * B, step=B)
        def _(off):
            pltpu.sync_copy(dst_hbm.at[pl.ds(sid * (EBT * B) + off, B)], didx)
            pltpu.sync_copy(ones_v, acc.at[didx], add=True)

    @pl.when(cid == 1)
    def _():
        @pl.loop(0, NBT * B, step=B)
        def _(off):
            pltpu.sync_copy(bx_hbm.at[pl.ds(sid * (NBT * B) + off, B)], didx)
            pltpu.sync_copy(ones_v, acc.at[didx], add=True)

    plsc.subcore_barrier()

    @pl.when(cid == 0)
    def _():
        pltpu.sync_copy(acc.at[stripe_slc], deg_out.at[stripe_slc])

    @pl.when(cid == 1)
    def _():
        pltpu.sync_copy(acc.at[stripe_slc], cnt_out.at[stripe_slc])


def _make_scatter(n_batches, sup):
    nsup = n_batches // sup

    @functools.partial(
        pl.kernel,
        compiler_params=_SC_PARAMS,
        out_type=(jax.ShapeDtypeStruct((NTAB, 128), jnp.float32),
                  jax.ShapeDtypeStruct((NTAB, 128), jnp.float32)),
        mesh=_MESH,
        scratch_types=[
            pltpu.VMEM((sup, B), jnp.int32),
            pltpu.VMEM((sup, B), jnp.int32),
            pltpu.VMEM((sup, B), jnp.int32),
            pltpu.VMEM((sup, B), jnp.int32),
            pltpu.VMEM((sup, B, C), jnp.float32),
            pltpu.VMEM((sup, B, C), jnp.float32),
            pltpu.VMEM_SHARED((NTAB, C), jnp.float32),
            pltpu.SemaphoreType.DMA,
            pltpu.SemaphoreType.DMA,
            pltpu.SemaphoreType.DMA,
        ],
    )
    def scat(g0_hbm, g1_hbm, g2_hbm, g3_hbm, dst_hbm, taba_hbm, tabb_hbm,
             zeros_hbm, outa, outb,
             sidx0, sidx1, didx0, didx1, rows0, rows1, acc,
             gsem0, gsem1, ssem):
        cid = lax.axis_index("c")
        sid = lax.axis_index("s")
        stripe_slc = pl.ds(sid * STRIPE, STRIPE)
        bbase = sid * n_batches
        gidx_hbm = (g0_hbm, g1_hbm, g2_hbm, g3_hbm)
        sidx = (sidx0, sidx1)
        didx = (didx0, didx1)
        rows = (rows0, rows1)
        gsem = (gsem0, gsem1)

        def load_idx(k, s, p):
            blk = pl.ds(bbase + s * sup, sup)
            pltpu.sync_copy(gidx_hbm[k % KH].at[blk], sidx[p])
            pltpu.sync_copy(dst_hbm.at[blk], didx[p])

        def fire_gathers(tab, p):
            for j in range(sup):
                pltpu.async_copy(tab.at[sidx[p].at[j]], rows[p].at[j], gsem[p])

        def wait_gathers(tab, p):
            for j in range(sup):
                pltpu.make_async_copy(tab.at[sidx[p].at[j]], rows[p].at[j],
                                      gsem[p]).wait()

        def scatter_adds(p):
            descs = [pltpu.async_copy(rows[p].at[j], acc.at[didx[p].at[j]],
                                      ssem, add=True) for j in range(sup)]
            for d in descs:
                d.wait()

        for k in range(K):
            tab = taba_hbm if k < KH else tabb_hbm
            out = outa if k < KH else outb

            @pl.when((k % 2) == cid)
            def _():
                pltpu.sync_copy(zeros_hbm, acc.at[stripe_slc])
                plsc.subcore_barrier()

                load_idx(k, 0, 0)
                fire_gathers(tab, 0)
                load_idx(k, 1, 1)
                fire_gathers(tab, 1)

                @pl.loop(0, nsup, step=2)
                def _(s):
                    i_last = (s + 2) >= nsup
                    wait_gathers(tab, 0)
                    scatter_adds(0)

                    @pl.when(jnp.logical_not(i_last))
                    def _():
                        load_idx(k, s + 2, 0)
                        fire_gathers(tab, 0)

                    wait_gathers(tab, 1)
                    scatter_adds(1)

                    @pl.when(jnp.logical_not(i_last))
                    def _():
                        load_idx(k, s + 3, 1)
                        fire_gathers(tab, 1)

                plsc.subcore_barrier()
                pltpu.sync_copy(acc.at[stripe_slc],
                                out.at[stripe_slc, pl.ds((k % KH) * C, C)])
                plsc.subcore_barrier()

    return scat


_s3 = _make_scatter(EBT, 3)
_s7 = _make_scatter(NBT, 2)


@functools.partial(
    pl.kernel,
    compiler_params=_SC_PARAMS,
    out_type=(jax.ShapeDtypeStruct((NBQ, 128), jnp.float32),
              jax.ShapeDtypeStruct((NBQ, 128), jnp.float32)),
    mesh=_MESH,
    scratch_types=[
        pltpu.VMEM((B,), jnp.int32),
        pltpu.VMEM((B,), jnp.int32),
        pltpu.VMEM((B, 128), jnp.float32),
        pltpu.VMEM((B, 128), jnp.float32),
        pltpu.VMEM((B, 128), jnp.float32),
        pltpu.VMEM((B, 128), jnp.float32),
        pltpu.SemaphoreType.DMA,
        pltpu.SemaphoreType.DMA,
    ],
)
def _s9(bx_hbm, taba_hbm, tabb_hbm, za_out, zb_out,
        ix0, ix1, ra0, ra1, rb0, rb1, sem0, sem1):
    wid = lax.axis_index("s") * 2 + lax.axis_index("c")
    base0 = wid * RPT32
    nb = RPT32 // B
    ix = (ix0, ix1)
    ra = (ra0, ra1)
    rb = (rb0, rb1)
    sem = (sem0, sem1)

    def fire(b, p):
        blk = pl.ds(base0 + b * B, B)
        pltpu.sync_copy(bx_hbm.at[blk], ix[p])
        pltpu.async_copy(taba_hbm.at[ix[p]], ra[p], sem[p])
        pltpu.async_copy(tabb_hbm.at[ix[p]], rb[p], sem[p])

    def drain(b, p):
        pltpu.make_async_copy(taba_hbm.at[ix[p]], ra[p], sem[p]).wait()
        pltpu.make_async_copy(tabb_hbm.at[ix[p]], rb[p], sem[p]).wait()
        blk = pl.ds(base0 + b * B, B)
        pltpu.sync_copy(ra[p], za_out.at[blk])
        pltpu.sync_copy(rb[p], zb_out.at[blk])

    fire(0, 0)

    @pl.loop(0, nb, step=2)
    def _(b):
        i_last = (b + 2) >= nb
        fire(b + 1, 1)
        drain(b, 0)

        @pl.when(jnp.logical_not(i_last))
        def _():
            fire(b + 2, 0)

        drain(b + 1, 1)


def _t9_body(za_ref, zb_ref, out_ref):
    out_ref[...] = jnp.concatenate([za_ref[...], zb_ref[:, :D_HID - 128]],
                                   axis=1)


def _t9(za, zb):
    return pl.pallas_call(
        _t9_body,
        grid=(125,),
        in_specs=[
            pl.BlockSpec((400, 128), lambda i: (i, 0)),
            pl.BlockSpec((400, 128), lambda i: (i, 0)),
        ],
        out_specs=pl.BlockSpec((400, D_HID), lambda i: (i, 0)),
        out_shape=jax.ShapeDtypeStruct((NB, D_HID), jnp.float32),
    )(za, zb)


def kernel(entity_feat, relation_embeddings, W_rel_in, b_rel_in,
           W_gcn1, W_gcn2, b_x, b_node_graph_index, edge_index):
    f32 = jnp.float32
    i32 = jnp.int32
    efp = jnp.pad(entity_feat, ((0, NTAB - N), (0, DEF - D_FEAT)))
    w1p = jnp.pad(W_gcn1, ((0, 0), (0, DP - D_HID)))
    w1top = jnp.pad(w1p[:D_FEAT], ((0, DEF - D_FEAT), (0, 0)))
    w1bot = w1p[D_FEAT:D_HID]
    w2p = jnp.pad(W_gcn2, ((0, DP - D_HID), (0, DP - D_HID)))
    pad_bx = jnp.full((NBQ - NB,), DUMMY, i32)
    bxp = jnp.concatenate([b_x.astype(i32), pad_bx])
    bgip = jnp.concatenate([b_node_graph_index.astype(i32),
                            jnp.zeros((NBQ - NB,), i32)])
    pad_e = jnp.full((EP - E,), DUMMY, i32)
    srcp = jnp.concatenate([edge_index[0].astype(i32), pad_e])
    dstb = jnp.concatenate([edge_index[1].astype(i32),
                            pad_e]).reshape(EP // B, B)
    src4 = [(srcp * 4 + kk).reshape(EP // B, B) for kk in range(KH)]
    ramp = jnp.arange(NBQ, dtype=i32) * 4
    ramp4 = [(ramp + kk).reshape(NBQ // B, B) for kk in range(KH)]
    bxb = bxp.reshape(NBQ // B, B)
    zeros_hbm = jnp.zeros((STRIPE, C), f32)
    ones_hbm = jnp.ones((B, C), f32)
    brel = b_rel_in.reshape(1, D_FEAT)

    rcwa, rcwb = _t0(relation_embeddings, W_rel_in, brel, w1bot)
    ef_g, s1a, s1b = _s1(bxp, bgip, efp, rcwa, rcwb)
    deg, cnt = _s3a(dstb.reshape(EP), bxp, ones_hbm, zeros_hbm)
    invd, invc = _t3(deg.reshape(NTAB // 4, 128), cnt.reshape(NTAB // 4, 128))
    sup1a, sup1b = _t2(ef_g, s1a, s1b, w1top)

    def edge_agg(supa, supb):
        va = supa.reshape(KH * NTAB, C)
        vb = supb.reshape(KH * NTAB, C)
        return _s3(src4[0], src4[1], src4[2], src4[3], dstb,
                   va, vb, zeros_hbm)

    agg1a, agg1b = edge_agg(sup1a, sup1b)
    sup2a, sup2b = _t4(agg1a, agg1b, sup1a, sup1b, invd, w2p)
    agg2a, agg2b = edge_agg(sup2a, sup2b)
    ea, eb = _t6(agg2a, agg2b, sup2a, sup2b, invd)
    sumsa, sumsb = _s7(ramp4[0], ramp4[1], ramp4[2], ramp4[3], bxb,
                       ea.reshape(KH * NBQ, C), eb.reshape(KH * NBQ, C),
                       zeros_hbm)
    taba, tabb = _t8(sumsa, sumsb, invc)
    za, zb = _s9(bxp, taba, tabb)
    return _t9(za, zb)


# speedup vs baseline: 1.3135x; 1.3135x over previous
import functools

import jax
import jax.numpy as jnp
from jax import lax
from jax.experimental import pallas as pl
from jax.experimental.pallas import tpu as pltpu
from jax.experimental.pallas import tpu_sc as plsc

N = 50000
NB = 50000
E = 800000
R = 1000
D_FEAT = 100
D_REL = 300
D_HID = 200

C = 32
K = 8
KH = 4
DP = K * C
DEF = 128
NTAB = 50176
NBQ = 57344
EP = 802816
DUMMY = 50000
B = 128
SUP = 2

STRIPE = NTAB // 16
RPT32 = NBQ // 32
EBT = EP // (16 * B)
NBT = NBQ // (16 * B)

_MESH = plsc.VectorSubcoreMesh(core_axis_name="c", subcore_axis_name="s")
_SC_PARAMS = pltpu.CompilerParams(use_tc_tiling_on_sc=False)
_RB = 512
_GRID = NTAB // _RB


def _t0_body(rel_ref, wr_ref, br_ref, w1b_ref, outa_ref, outb_ref):
    rc = jnp.dot(rel_ref[...], wr_ref[...], preferred_element_type=jnp.float32)
    rc = jnp.maximum(rc + br_ref[...], 0.0)
    s = jnp.dot(rc, w1b_ref[...], preferred_element_type=jnp.float32)
    outa_ref[...] = s[:, :128]
    outb_ref[...] = s[:, 128:]


def _t0(rel, wr, br, w1b):
    return pl.pallas_call(
        _t0_body,
        out_shape=(jax.ShapeDtypeStruct((R, 128), jnp.float32),
                   jax.ShapeDtypeStruct((R, 128), jnp.float32)),
    )(rel, wr, br, w1b)


def _t2_body(ef_ref, s1a_ref, s1b_ref, w_ref, outa_ref, outb_ref):
    s = jnp.dot(ef_ref[...], w_ref[...], preferred_element_type=jnp.float32)
    outa_ref[...] = s[:, :128] + s1a_ref[...]
    outb_ref[...] = s[:, 128:] + s1b_ref[...]


def _t2(ef, s1a, s1b, w):
    return pl.pallas_call(
        _t2_body,
        grid=(_GRID,),
        in_specs=[
            pl.BlockSpec((_RB, DEF), lambda i: (i, 0)),
            pl.BlockSpec((_RB, 128), lambda i: (i, 0)),
            pl.BlockSpec((_RB, 128), lambda i: (i, 0)),
            pl.BlockSpec((DEF, DP), lambda i: (0, 0)),
        ],
        out_specs=(pl.BlockSpec((_RB, 128), lambda i: (i, 0)),
                   pl.BlockSpec((_RB, 128), lambda i: (i, 0))),
        out_shape=(jax.ShapeDtypeStruct((NTAB, 128), jnp.float32),
                   jax.ShapeDtypeStruct((NTAB, 128), jnp.float32)),
    )(ef, s1a, s1b, w)


def _t3_body(deg_ref, cnt_ref, invd_ref, invc_ref):
    def unpack(x):
        parts = [jnp.broadcast_to(x[:, 32 * a:32 * a + 1], (_RB // 4, 128))
                 for a in range(4)]
        return jnp.stack(parts, axis=1).reshape(_RB, 128)

    invd_ref[...] = 1.0 / (unpack(deg_ref[...]) + 1.0)
    invc_ref[...] = 1.0 / jnp.maximum(unpack(cnt_ref[...]), 1.0)


def _t3(deg4, cnt4):
    return pl.pallas_call(
        _t3_body,
        grid=(_GRID,),
        in_specs=[
            pl.BlockSpec((_RB // 4, 128), lambda i: (i, 0)),
            pl.BlockSpec((_RB // 4, 128), lambda i: (i, 0)),
        ],
        out_specs=(pl.BlockSpec((_RB, 128), lambda i: (i, 0)),
                   pl.BlockSpec((_RB, 128), lambda i: (i, 0))),
        out_shape=(jax.ShapeDtypeStruct((NTAB, 128), jnp.float32),
                   jax.ShapeDtypeStruct((NTAB, 128), jnp.float32)),
    )(deg4, cnt4)


def _t4_body(agga_ref, aggb_ref, supa_ref, supb_ref, inv_ref, w_ref,
             outa_ref, outb_ref):
    inv = inv_ref[...]
    ha = jnp.maximum((agga_ref[...] + supa_ref[...]) * inv, 0.0)
    hb = jnp.maximum((aggb_ref[...] + supb_ref[...]) * inv, 0.0)
    h = jnp.concatenate([ha, hb], axis=1)
    s2 = jnp.dot(h, w_ref[...], preferred_element_type=jnp.float32)
    outa_ref[...] = s2[:, :128]
    outb_ref[...] = s2[:, 128:]


def _t4(agga, aggb, supa, supb, inv, w):
    bs = pl.BlockSpec((_RB, 128), lambda i: (i, 0))
    return pl.pallas_call(
        _t4_body,
        grid=(_GRID,),
        in_specs=[bs, bs, bs, bs, bs, pl.BlockSpec((DP, DP), lambda i: (0, 0))],
        out_specs=(bs, bs),
        out_shape=(jax.ShapeDtypeStruct((NTAB, 128), jnp.float32),
                   jax.ShapeDtypeStruct((NTAB, 128), jnp.float32)),
    )(agga, aggb, supa, supb, inv, w)


def _t6_body(agga_ref, aggb_ref, supa_ref, supb_ref, inv_ref,
             outa_ref, outb_ref):
    inv = inv_ref[...]
    outa_ref[...] = (agga_ref[...] + supa_ref[...]) * inv
    outb_ref[...] = (aggb_ref[...] + supb_ref[...]) * inv


def _t6(agga, aggb, supa, supb, inv):
    bs = pl.BlockSpec((_RB, 128), lambda i: (i, 0))
    return pl.pallas_call(
        _t6_body,
        grid=(_GRID,),
        in_specs=[bs, bs, bs, bs, bs],
        out_specs=(bs, bs),
        out_shape=(jax.ShapeDtypeStruct((NBQ, 128), jnp.float32),
                   jax.ShapeDtypeStruct((NBQ, 128), jnp.float32)),
    )(agga, aggb, supa, supb, inv)


def _t8_body(sumsa_ref, sumsb_ref, inv_ref, outa_ref, outb_ref):
    inv = inv_ref[...]
    outa_ref[...] = sumsa_ref[...] * inv
    outb_ref[...] = sumsb_ref[...] * inv


def _t8(sumsa, sumsb, inv):
    bs = pl.BlockSpec((_RB, 128), lambda i: (i, 0))
    return pl.pallas_call(
        _t8_body,
        grid=(_GRID,),
        in_specs=[bs, bs, bs],
        out_specs=(bs, bs),
        out_shape=(jax.ShapeDtypeStruct((NTAB, 128), jnp.float32),
                   jax.ShapeDtypeStruct((NTAB, 128), jnp.float32)),
    )(sumsa, sumsb, inv)


@functools.partial(
    pl.kernel,
    compiler_params=_SC_PARAMS,
    out_type=(jax.ShapeDtypeStruct((NBQ, DEF), jnp.float32),
              jax.ShapeDtypeStruct((NBQ, 128), jnp.float32),
              jax.ShapeDtypeStruct((NBQ, 128), jnp.float32)),
    mesh=_MESH,
    scratch_types=[
        pltpu.VMEM((B,), jnp.int32),
        pltpu.VMEM((B,), jnp.int32),
        pltpu.VMEM((B,), jnp.int32),
        pltpu.VMEM((B,), jnp.int32),
        pltpu.VMEM((B, DEF), jnp.float32),
        pltpu.VMEM((B, DEF), jnp.float32),
        pltpu.VMEM((B, 128), jnp.float32),
        pltpu.VMEM((B, 128), jnp.float32),
        pltpu.VMEM((B, 128), jnp.float32),
        pltpu.VMEM((B, 128), jnp.float32),
        pltpu.SemaphoreType.DMA,
        pltpu.SemaphoreType.DMA,
    ],
)
def _s1(bx_hbm, bgi_hbm, ef_hbm, rcwa_hbm, rcwb_hbm, ef_out, s1a_out, s1b_out,
        ix0, ix1, ig0, ig1, re0, re1, ra0, ra1, rb0, rb1, sem0, sem1):
    wid = lax.axis_index("s") * 2 + lax.axis_index("c")
    base0 = wid * RPT32
    nb = RPT32 // B
    ix = (ix0, ix1)
    ig = (ig0, ig1)
    re = (re0, re1)
    ra = (ra0, ra1)
    rb = (rb0, rb1)
    sem = (sem0, sem1)

    def fire(b, p):
        blk = pl.ds(base0 + b * B, B)
        pltpu.sync_copy(bx_hbm.at[blk], ix[p])
        pltpu.sync_copy(bgi_hbm.at[blk], ig[p])
        pltpu.async_copy(ef_hbm.at[ix[p]], re[p], sem[p])
        pltpu.async_copy(rcwa_hbm.at[ig[p]], ra[p], sem[p])
        pltpu.async_copy(rcwb_hbm.at[ig[p]], rb[p], sem[p])

    def drain(b, p):
        pltpu.make_async_copy(ef_hbm.at[ix[p]], re[p], sem[p]).wait()
        pltpu.make_async_copy(rcwa_hbm.at[ig[p]], ra[p], sem[p]).wait()
        pltpu.make_async_copy(rcwb_hbm.at[ig[p]], rb[p], sem[p]).wait()
        blk = pl.ds(base0 + b * B, B)
        pltpu.sync_copy(re[p], ef_out.at[blk])
        pltpu.sync_copy(ra[p], s1a_out.at[blk])
        pltpu.sync_copy(rb[p], s1b_out.at[blk])

    fire(0, 0)

    @pl.loop(0, nb, step=2)
    def _(b):
        i_last = (b + 2) >= nb
        fire(b + 1, 1)
        drain(b, 0)

        @pl.when(jnp.logical_not(i_last))
        def _():
            fire(b + 2, 0)

        drain(b + 1, 1)


@functools.partial(
    pl.kernel,
    compiler_params=_SC_PARAMS,
    out_type=(jax.ShapeDtypeStruct((NTAB, C), jnp.float32),
              jax.ShapeDtypeStruct((NTAB, C), jnp.float32)),
    mesh=_MESH,
    scratch_types=[
        pltpu.VMEM((B,), jnp.int32),
        pltpu.VMEM((B, C), jnp.float32),
        pltpu.VMEM_SHARED((NTAB, C), jnp.float32),
    ],
)
def _s3a(dst_hbm, bx_hbm, ones_hbm, zeros_hbm, deg_out, cnt_out,
         didx, ones_v, acc):
    cid = lax.axis_index("c")
    sid = lax.axis_index("s")
    stripe_slc = pl.ds(sid * STRIPE, STRIPE)
    pltpu.sync_copy(ones_hbm, ones_v)
    pltpu.sync_copy(zeros_hbm, acc.at[stripe_slc])
    plsc.subcore_barrier()

    @pl.when(cid == 0)
    def _():
        @pl.loop(0, EBT * B, step=B)
        def _(off):
            pltpu.sync_copy(dst_hbm.at[pl.ds(sid * (EBT * B) + off, B)], didx)
            pltpu.sync_copy(ones_v, acc.at[didx], add=True)

    @pl.when(cid == 1)
    def _():
        @pl.loop(0, NBT * B, step=B)
        def _(off):
            pltpu.sync_copy(bx_hbm.at[pl.ds(sid * (NBT * B) + off, B)], didx)
            pltpu.sync_copy(ones_v, acc.at[didx], add=True)

    plsc.subcore_barrier()

    @pl.when(cid == 0)
    def _():
        pltpu.sync_copy(acc.at[stripe_slc], deg_out.at[stripe_slc])

    @pl.when(cid == 1)
    def _():
        pltpu.sync_copy(acc.at[stripe_slc], cnt_out.at[stripe_slc])


def _make_scatter(n_batches, sup):
    nsup = n_batches // sup

    @functools.partial(
        pl.kernel,
        compiler_params=_SC_PARAMS,
        out_type=(jax.ShapeDtypeStruct((NTAB, 128), jnp.float32),
                  jax.ShapeDtypeStruct((NTAB, 128), jnp.float32)),
        mesh=_MESH,
        scratch_types=[
            pltpu.VMEM((sup, 2, B), jnp.int32),
            pltpu.VMEM((sup, 2, B), jnp.int32),
            pltpu.VMEM((sup, B, C), jnp.float32),
            pltpu.VMEM((sup, B, C), jnp.float32),
            pltpu.VMEM_SHARED((NTAB, C), jnp.float32),
            pltpu.SemaphoreType.DMA,
            pltpu.SemaphoreType.DMA,
            pltpu.SemaphoreType.DMA,
        ],
    )
    def scat(g0_hbm, g1_hbm, g2_hbm, g3_hbm, taba_hbm, tabb_hbm,
             zeros_hbm, outa, outb,
             cidx0, cidx1, rows0, rows1, acc,
             gsem0, gsem1, ssem):
        cid = lax.axis_index("c")
        sid = lax.axis_index("s")
        stripe_slc = pl.ds(sid * STRIPE, STRIPE)
        bbase = sid * n_batches
        gidx_hbm = (g0_hbm, g1_hbm, g2_hbm, g3_hbm)
        cidx = (cidx0, cidx1)
        rows = (rows0, rows1)
        gsem = (gsem0, gsem1)

        def load_idx(k, s, p):
            blk = pl.ds(bbase + s * sup, sup)
            pltpu.sync_copy(gidx_hbm[k % KH].at[blk], cidx[p])

        def fire_gathers(tab, p):
            for j in range(sup):
                pltpu.async_copy(tab.at[cidx[p].at[j, 0]], rows[p].at[j],
                                 gsem[p])

        def wait_gathers(tab, p):
            for j in range(sup):
                pltpu.make_async_copy(tab.at[cidx[p].at[j, 0]], rows[p].at[j],
                                      gsem[p]).wait()

        def scatter_adds(p):
            descs = [pltpu.async_copy(rows[p].at[j],
                                      acc.at[cidx[p].at[j, 1]],
                                      ssem, add=True) for j in range(sup)]
            for d in descs:
                d.wait()

        for k in range(K):
            tab = taba_hbm if k < KH else tabb_hbm
            out = outa if k < KH else outb

            @pl.when((k % 2) == cid)
            def _():
                pltpu.sync_copy(zeros_hbm, acc.at[stripe_slc])
                plsc.subcore_barrier()

                load_idx(k, 0, 0)
                fire_gathers(tab, 0)
                load_idx(k, 1, 1)
                fire_gathers(tab, 1)

                @pl.loop(0, nsup, step=2)
                def _(s):
                    i_last = (s + 2) >= nsup
                    wait_gathers(tab, 0)
                    scatter_adds(0)

                    @pl.when(jnp.logical_not(i_last))
                    def _():
                        load_idx(k, s + 2, 0)
                        fire_gathers(tab, 0)

                    wait_gathers(tab, 1)
                    scatter_adds(1)

                    @pl.when(jnp.logical_not(i_last))
                    def _():
                        load_idx(k, s + 3, 1)
                        fire_gathers(tab, 1)

                plsc.subcore_barrier()
                pltpu.sync_copy(acc.at[stripe_slc],
                                out.at[stripe_slc, pl.ds((k % KH) * C, C)])
                plsc.subcore_barrier()

    return scat


_s3 = _make_scatter(EBT, 2)
_s7 = _make_scatter(NBT, 2)


@functools.partial(
    pl.kernel,
    compiler_params=_SC_PARAMS,
    out_type=(jax.ShapeDtypeStruct((NBQ, 128), jnp.float32),
              jax.ShapeDtypeStruct((NBQ, 128), jnp.float32)),
    mesh=_MESH,
    scratch_types=[
        pltpu.VMEM((B,), jnp.int32),
        pltpu.VMEM((B,), jnp.int32),
        pltpu.VMEM((B, 128), jnp.float32),
        pltpu.VMEM((B, 128), jnp.float32),
        pltpu.VMEM((B, 128), jnp.float32),
        pltpu.VMEM((B, 128), jnp.float32),
        pltpu.SemaphoreType.DMA,
        pltpu.SemaphoreType.DMA,
    ],
)
def _s9(bx_hbm, taba_hbm, tabb_hbm, za_out, zb_out,
        ix0, ix1, ra0, ra1, rb0, rb1, sem0, sem1):
    wid = lax.axis_index("s") * 2 + lax.axis_index("c")
    base0 = wid * RPT32
    nb = RPT32 // B
    ix = (ix0, ix1)
    ra = (ra0, ra1)
    rb = (rb0, rb1)
    sem = (sem0, sem1)

    def fire(b, p):
        blk = pl.ds(base0 + b * B, B)
        pltpu.sync_copy(bx_hbm.at[blk], ix[p])
        pltpu.async_copy(taba_hbm.at[ix[p]], ra[p], sem[p])
        pltpu.async_copy(tabb_hbm.at[ix[p]], rb[p], sem[p])

    def drain(b, p):
        pltpu.make_async_copy(taba_hbm.at[ix[p]], ra[p], sem[p]).wait()
        pltpu.make_async_copy(tabb_hbm.at[ix[p]], rb[p], sem[p]).wait()
        blk = pl.ds(base0 + b * B, B)
        pltpu.sync_copy(ra[p], za_out.at[blk])
        pltpu.sync_copy(rb[p], zb_out.at[blk])

    fire(0, 0)

    @pl.loop(0, nb, step=2)
    def _(b):
        i_last = (b + 2) >= nb
        fire(b + 1, 1)
        drain(b, 0)

        @pl.when(jnp.logical_not(i_last))
        def _():
            fire(b + 2, 0)

        drain(b + 1, 1)


def _t9_body(za_ref, zb_ref, out_ref):
    out_ref[...] = jnp.concatenate([za_ref[...], zb_ref[:, :D_HID - 128]],
                                   axis=1)


def _t9(za, zb):
    return pl.pallas_call(
        _t9_body,
        grid=(125,),
        in_specs=[
            pl.BlockSpec((400, 128), lambda i: (i, 0)),
            pl.BlockSpec((400, 128), lambda i: (i, 0)),
        ],
        out_specs=pl.BlockSpec((400, D_HID), lambda i: (i, 0)),
        out_shape=jax.ShapeDtypeStruct((NB, D_HID), jnp.float32),
    )(za, zb)


def kernel(entity_feat, relation_embeddings, W_rel_in, b_rel_in,
           W_gcn1, W_gcn2, b_x, b_node_graph_index, edge_index):
    f32 = jnp.float32
    i32 = jnp.int32
    efp = jnp.pad(entity_feat, ((0, NTAB - N), (0, DEF - D_FEAT)))
    w1p = jnp.pad(W_gcn1, ((0, 0), (0, DP - D_HID)))
    w1top = jnp.pad(w1p[:D_FEAT], ((0, DEF - D_FEAT), (0, 0)))
    w1bot = w1p[D_FEAT:D_HID]
    w2p = jnp.pad(W_gcn2, ((0, DP - D_HID), (0, DP - D_HID)))
    pad_bx = jnp.full((NBQ - NB,), DUMMY, i32)
    bxp = jnp.concatenate([b_x.astype(i32), pad_bx])
    bgip = jnp.concatenate([b_node_graph_index.astype(i32),
                            jnp.zeros((NBQ - NB,), i32)])
    pad_e = jnp.full((EP - E,), DUMMY, i32)
    srcp = jnp.concatenate([edge_index[0].astype(i32), pad_e])
    dstb = jnp.concatenate([edge_index[1].astype(i32),
                            pad_e]).reshape(EP // B, B)
    src4 = [jnp.stack([(srcp * 4 + kk).reshape(EP // B, B), dstb], axis=1)
            for kk in range(KH)]
    ramp = jnp.arange(NBQ, dtype=i32) * 4
    bxb = bxp.reshape(NBQ // B, B)
    ramp4 = [jnp.stack([(ramp + kk).reshape(NBQ // B, B), bxb], axis=1)
             for kk in range(KH)]
    zeros_hbm = jnp.zeros((STRIPE, C), f32)
    ones_hbm = jnp.ones((B, C), f32)
    brel = b_rel_in.reshape(1, D_FEAT)

    rcwa, rcwb = _t0(relation_embeddings, W_rel_in, brel, w1bot)
    ef_g, s1a, s1b = _s1(bxp, bgip, efp, rcwa, rcwb)
    deg, cnt = _s3a(dstb.reshape(EP), bxp, ones_hbm, zeros_hbm)
    invd, invc = _t3(deg.reshape(NTAB // 4, 128), cnt.reshape(NTAB // 4, 128))
    sup1a, sup1b = _t2(ef_g, s1a, s1b, w1top)

    def edge_agg(supa, supb):
        va = supa.reshape(KH * NTAB, C)
        vb = supb.reshape(KH * NTAB, C)
        return _s3(src4[0], src4[1], src4[2], src4[3],
                   va, vb, zeros_hbm)

    agg1a, agg1b = edge_agg(sup1a, sup1b)
    sup2a, sup2b = _t4(agg1a, agg1b, sup1a, sup1b, invd, w2p)
    agg2a, agg2b = edge_agg(sup2a, sup2b)
    ea, eb = _t6(agg2a, agg2b, sup2a, sup2b, invd)
    sumsa, sumsb = _s7(ramp4[0], ramp4[1], ramp4[2], ramp4[3],
                       ea.reshape(KH * NBQ, C), eb.reshape(KH * NBQ, C),
                       zeros_hbm)
    taba, tabb = _t8(sumsa, sumsb, invc)
    za, zb = _s9(bxp, taba, tabb)
    return _t9(za, zb)
